# Initial kernel scaffold; baseline (speedup 1.0000x reference)
import jax
import jax.numpy as jnp
from jax.experimental import pallas as pl


def kernel(center, outside, negative, emb_center, emb_outside):
    raise NotImplementedError("write your pallas kernel here")


# speedup vs baseline: 5.4066x; 5.4066x over previous
import functools

import jax
import jax.numpy as jnp
from jax import lax
from jax.experimental import pallas as pl
from jax.experimental.pallas import tpu as pltpu
from jax.experimental.pallas import tpu_sc as plsc

V, E, B, K = 1000000, 64, 16384, 20
NC, NS = 2, 16
NW = NC * NS
S = B // NW
C = 32
NCH = S // C
NIR = C * K // 128
NROW = S * K // 128
EV = E // 16


def _sc_partials(center, outside, neg2d, emb_center, emb_outside):
  mesh = plsc.VectorSubcoreMesh(core_axis_name="c", subcore_axis_name="s")

  @functools.partial(
      pl.kernel, mesh=mesh,
      out_type=jax.ShapeDtypeStruct((B, 32), jnp.float32),
      compiler_params=pltpu.CompilerParams(use_tc_tiling_on_sc=False),
      scratch_types=[
          pltpu.VMEM((S,), jnp.int32),
          pltpu.VMEM((S,), jnp.int32),
          pltpu.VMEM((NROW, 128), jnp.int32),
          pltpu.VMEM((C, E), jnp.float32),
          pltpu.VMEM((C, E), jnp.float32),
          pltpu.VMEM((C, E), jnp.float32),
          pltpu.VMEM((C, E), jnp.float32),
          pltpu.VMEM((C * K, E), jnp.float32),
          pltpu.VMEM((C * K, E), jnp.float32),
          pltpu.VMEM((S, 32), jnp.float32),
          pltpu.SemaphoreType.DMA,
          pltpu.SemaphoreType.DMA,
      ])
  def k(center_hbm, outside_hbm, neg_hbm, embc_hbm, embo_hbm, out_hbm,
        cidx, oidx, nidx, crows0, crows1, orows0, orows1, nrows0, nrows1,
        outv, sem0, sem1):
    crows = (crows0, crows1)
    orows = (orows0, orows1)
    nrows = (nrows0, nrows1)
    sems = (sem0, sem1)
    wid = lax.axis_index("s") * NC + lax.axis_index("c")
    base = wid * S

    pltpu.sync_copy(center_hbm.at[pl.ds(base, S)], cidx)
    pltpu.sync_copy(outside_hbm.at[pl.ds(base, S)], oidx)
    pltpu.sync_copy(neg_hbm.at[pl.ds(wid * NROW, NROW), :], nidx)

    def issue(g, p):
      pltpu.async_copy(embc_hbm.at[cidx.at[pl.ds(g * C, C)]], crows[p], sems[p])
      pltpu.async_copy(embo_hbm.at[oidx.at[pl.ds(g * C, C)]], orows[p], sems[p])
      for j in range(NIR):
        pltpu.async_copy(embo_hbm.at[nidx.at[g * NIR + j]],
                         nrows[p].at[pl.ds(j * 128, 128)], sems[p])

    def wait(p):
      pltpu.make_async_copy(embc_hbm.at[pl.ds(0, C)], crows[p], sems[p]).wait()
      pltpu.make_async_copy(embo_hbm.at[pl.ds(0, C)], orows[p], sems[p]).wait()
      for j in range(NIR):
        pltpu.make_async_copy(embo_hbm.at[pl.ds(0, 128)],
                              nrows[p].at[pl.ds(j * 128, 128)],
                              sems[p]).wait()

    def compute(g, p):
      cr, orr, nr = crows[p], orows[p], nrows[p]

      def body(b, carry):
        cs = [cr[b, pl.ds(16 * j, 16)] for j in range(EV)]
        acc_o = cs[0] * orr[b, pl.ds(0, 16)]
        for j in range(1, EV):
          acc_o = acc_o + cs[j] * orr[b, pl.ds(16 * j, 16)]
        acc_n = None
        for j in range(EV):
          s = nr[b * K, pl.ds(16 * j, 16)]
          for kk in range(1, K):
            s = s + nr[b * K + kk, pl.ds(16 * j, 16)]
          t = s * cs[j]
          acc_n = t if acc_n is None else acc_n + t
        row = g * C + b
        outv[row, pl.ds(0, 16)] = acc_o
        outv[row, pl.ds(16, 16)] = acc_n
        return carry

      lax.fori_loop(0, C, body, 0)

    issue(0, 0)

    def outer(gp, carry):
      for lane in range(2):
        g = gp * 2 + lane

        @pl.when(g + 1 < NCH)
        def _():
          issue(g + 1, (lane + 1) % 2)

        wait(lane)
        compute(g, lane)
      return carry

    lax.fori_loop(0, NCH // 2, outer, 0)
    pltpu.sync_copy(outv, out_hbm.at[pl.ds(base, S), :])

  return k(center, outside, neg2d, emb_center, emb_outside)


def _logsig(x):
  return jnp.minimum(x, 0.0) - jnp.log1p(jnp.exp(-jnp.abs(x)))


def _finish_body(p_ref, o_ref):
  x = p_ref[...]
  uovc = jnp.sum(x[:, 0:16], axis=1)
  nd = jnp.sum(x[:, 16:32], axis=1)
  loss = _logsig(uovc) + _logsig(-nd)
  o_ref[...] = jnp.broadcast_to(-jnp.mean(loss), (1, 1))


def kernel(center, outside, negative, emb_center, emb_outside):
  c = center.reshape(B).astype(jnp.int32)
  o = outside.reshape(B).astype(jnp.int32)
  n = negative.reshape(B * K // 128, 128).astype(jnp.int32)
  parts = _sc_partials(c, o, n, emb_center, emb_outside)
  out = pl.pallas_call(
      _finish_body,
      out_shape=jax.ShapeDtypeStruct((1, 1), jnp.float32))(parts)
  return out[0, 0]
